# Initial kernel scaffold; baseline (speedup 1.0000x reference)
import jax
import jax.numpy as jnp
from jax.experimental import pallas as pl


def kernel(x, edge_index, weight):
    raise NotImplementedError("write your pallas kernel here")


# speedup vs baseline: 16.0667x; 16.0667x over previous
import functools

import jax
import jax.numpy as jnp
from jax import lax
from jax.experimental import pallas as pl
from jax.experimental.pallas import tpu as pltpu
from jax.experimental.pallas import tpu_sc as plsc

N = 10000
E = 320000
D = 128

NC, NS = 2, 16
NW = NC * NS
NPAD = 10240
EPW = E // NW
K = 80
ROWS_PT = NPAD // NS

_mesh = plsc.VectorSubcoreMesh(
    core_axis_name="c", subcore_axis_name="s", num_cores=NC, num_subcores=NS)


def _hist_body(row_hbm, ones_hbm, zeros_hbm, out_hbm, idx_v, ones_v, zero_v,
               deg_s, sem):
    c = lax.axis_index("c")
    s = lax.axis_index("s")
    w = c * NS + s

    pltpu.sync_copy(ones_hbm, ones_v)
    pltpu.sync_copy(zeros_hbm, zero_v)
    for j in range(4):
        pltpu.sync_copy(zero_v, deg_s.at[pl.ds(s * ROWS_PT + j * (ROWS_PT // 4),
                                               ROWS_PT // 4)])
    plsc.subcore_barrier()

    def body(i, _):
        base = w * EPW + i * K
        pltpu.sync_copy(row_hbm.at[pl.ds(base, K)], idx_v)
        pltpu.sync_copy(ones_v, deg_s.at[idx_v], add=True)
        return 0

    lax.fori_loop(0, EPW // K, body, 0)
    plsc.subcore_barrier()

    pltpu.sync_copy(deg_s.at[pl.ds(s * ROWS_PT, ROWS_PT)],
                    out_hbm.at[c, pl.ds(s * ROWS_PT, ROWS_PT)])


_hist_call = pl.kernel(
    _hist_body,
    out_type=jax.ShapeDtypeStruct((NC, NPAD, 128), jnp.float32),
    mesh=_mesh,
    scratch_types=[
        pltpu.VMEM((K,), jnp.int32),
        pltpu.VMEM((K, 128), jnp.float32),
        pltpu.VMEM((ROWS_PT // 4, 128), jnp.float32),
        pltpu.VMEM_SHARED((NPAD, 128), jnp.float32),
        pltpu.SemaphoreType.DMA,
    ],
)


def _hist(row):
    ones = jnp.ones((K, 128), jnp.float32)
    zeros = jnp.zeros((ROWS_PT // 4, 128), jnp.float32)
    return _hist_call(row, ones, zeros)


def _agg_body(g_hbm, row_hbm, col_hbm, out_hbm, cidx_v, ridx_v, rows_v, acc_s,
              sem):
    c = lax.axis_index("c")
    s = lax.axis_index("s")
    w = c * NS + s

    def zrow(i, _):
        def zcol(j, _):
            rows_v[i, pl.ds(j * 16, 16)] = jnp.zeros((16,), jnp.float32)
            return 0
        lax.fori_loop(0, D // 16, zcol, 0)
        return 0

    lax.fori_loop(0, K, zrow, 0)
    for j in range(ROWS_PT // K):
        pltpu.sync_copy(rows_v, acc_s.at[pl.ds(s * ROWS_PT + j * K, K)])
    plsc.subcore_barrier()

    def body(i, _):
        base = w * EPW + i * K
        pltpu.sync_copy(col_hbm.at[pl.ds(base, K)], cidx_v)
        pltpu.sync_copy(row_hbm.at[pl.ds(base, K)], ridx_v)
        pltpu.async_copy(g_hbm.at[cidx_v], rows_v, sem).wait()
        pltpu.sync_copy(rows_v, acc_s.at[ridx_v], add=True)
        return 0

    lax.fori_loop(0, EPW // K, body, 0)
    plsc.subcore_barrier()

    for j in range(ROWS_PT // K):
        r0 = s * ROWS_PT + j * K
        pltpu.sync_copy(acc_s.at[pl.ds(r0, K)], out_hbm.at[c, pl.ds(r0, K)])


_aggregate = pl.kernel(
    _agg_body,
    out_type=jax.ShapeDtypeStruct((NC, NPAD, D), jnp.float32),
    mesh=_mesh,
    scratch_types=[
        pltpu.VMEM((K,), jnp.int32),
        pltpu.VMEM((K,), jnp.int32),
        pltpu.VMEM((K, D), jnp.float32),
        pltpu.VMEM_SHARED((NPAD, D), jnp.float32),
        pltpu.SemaphoreType.DMA,
    ],
)


_RB = 2048


def _linear_body(x_ref, w_ref, degp_ref, g_ref, dis_ref):
    deg = degp_ref[0, :, 0:1] + degp_ref[1, :, 0:1] + 2.0
    dis = lax.rsqrt(deg)
    h = jnp.dot(x_ref[...], w_ref[...], preferred_element_type=jnp.float32)
    g_ref[...] = dis * h
    dis_ref[...] = dis


def _linear(x_pad, weight, deg_part):
    return pl.pallas_call(
        _linear_body,
        grid=(NPAD // _RB,),
        in_specs=[
            pl.BlockSpec((_RB, D), lambda i: (i, 0)),
            pl.BlockSpec((D, D), lambda i: (0, 0)),
            pl.BlockSpec((NC, _RB, 128), lambda i: (0, i, 0)),
        ],
        out_specs=[
            pl.BlockSpec((_RB, D), lambda i: (i, 0)),
            pl.BlockSpec((_RB, 1), lambda i: (i, 0)),
        ],
        out_shape=[
            jax.ShapeDtypeStruct((NPAD, D), jnp.float32),
            jax.ShapeDtypeStruct((NPAD, 1), jnp.float32),
        ],
    )(x_pad, weight, deg_part)


def _finish_body(acc_ref, g_ref, dis_ref, o_ref):
    acc = acc_ref[0] + acc_ref[1]
    o_ref[...] = jnp.maximum(dis_ref[...] * (acc + 2.0 * g_ref[...]), 0.0)


def _finish(acc, g, dis):
    return pl.pallas_call(
        _finish_body,
        grid=(NPAD // _RB,),
        in_specs=[
            pl.BlockSpec((NC, _RB, D), lambda i: (0, i, 0)),
            pl.BlockSpec((_RB, D), lambda i: (i, 0)),
            pl.BlockSpec((_RB, 1), lambda i: (i, 0)),
        ],
        out_specs=pl.BlockSpec((_RB, D), lambda i: (i, 0)),
        out_shape=jax.ShapeDtypeStruct((NPAD, D), jnp.float32),
    )(acc, g, dis)


def kernel(x, edge_index, weight):
    row = edge_index[0]
    col = edge_index[1]
    x_pad = jnp.pad(x, ((0, NPAD - N), (0, 0)))
    deg_part = _hist(row)
    g, dis = _linear(x_pad, weight, deg_part)
    acc = _aggregate(g, row, col)
    out = _finish(acc, g, dis)
    return out[:N]


# speedup vs baseline: 23.5313x; 1.4646x over previous
import jax
import jax.numpy as jnp
from jax import lax
from jax.experimental import pallas as pl
from jax.experimental.pallas import tpu as pltpu
from jax.experimental.pallas import tpu_sc as plsc

N = 10000
E = 320000
D = 128

NC, NS = 2, 16
NW = NC * NS
NPAD = 10240
EPW = E // NW
K = 80
NCH = EPW // K
ROWS_PT = NPAD // NS
APT = NPAD // NS

_mesh = plsc.VectorSubcoreMesh(
    core_axis_name="c", subcore_axis_name="s", num_cores=NC, num_subcores=NS)


def _hist_body(row_hbm, ones_hbm, zeros_hbm, out_hbm, idx_v, ones_v, zero_v,
               deg_s, sem):
    c = lax.axis_index("c")
    s = lax.axis_index("s")
    w = c * NS + s

    pltpu.sync_copy(ones_hbm, ones_v)
    pltpu.sync_copy(zeros_hbm, zero_v)
    for j in range(4):
        pltpu.sync_copy(zero_v, deg_s.at[pl.ds(s * ROWS_PT + j * (ROWS_PT // 4),
                                               ROWS_PT // 4)])
    plsc.subcore_barrier()

    def body(i, _):
        base = w * EPW + i * K
        pltpu.sync_copy(row_hbm.at[pl.ds(base, K)], idx_v)
        pltpu.sync_copy(ones_v, deg_s.at[idx_v], add=True)
        return 0

    lax.fori_loop(0, NCH, body, 0)
    plsc.subcore_barrier()

    pltpu.sync_copy(deg_s.at[pl.ds(s * ROWS_PT, ROWS_PT)],
                    out_hbm.at[c, pl.ds(s * ROWS_PT, ROWS_PT)])


_hist_call = pl.kernel(
    _hist_body,
    out_type=jax.ShapeDtypeStruct((NC, NPAD, 128), jnp.float32),
    mesh=_mesh,
    scratch_types=[
        pltpu.VMEM((K,), jnp.int32),
        pltpu.VMEM((K, 128), jnp.float32),
        pltpu.VMEM((ROWS_PT // 4, 128), jnp.float32),
        pltpu.VMEM_SHARED((NPAD, 128), jnp.float32),
        pltpu.SemaphoreType.DMA,
    ],
)


def _hist(row):
    ones = jnp.ones((K, 128), jnp.float32)
    zeros = jnp.zeros((ROWS_PT // 4, 128), jnp.float32)
    return _hist_call(row, ones, zeros)


def _agg_body(g_hbm, col_hbm, row3_hbm, zeros_hbm, out_hbm, cidx_buf, ridx_buf,
              rows_a, rows_b, acc_s, sem_ga, sem_gb, sem_sa, sem_sb):
    c = lax.axis_index("c")
    s = lax.axis_index("s")
    w = c * NS + s

    pltpu.sync_copy(zeros_hbm, acc_s.at[pl.ds(s * APT, APT)])
    pltpu.sync_copy(col_hbm.at[pl.ds(w * EPW, EPW)], cidx_buf)
    pltpu.sync_copy(row3_hbm.at[w], ridx_buf)
    plsc.subcore_barrier()

    def g_src(i):
        return g_hbm.at[cidx_buf.at[pl.ds(i * K, K)]]

    def issue_gather(i, buf, sem):
        pltpu.async_copy(g_src(i), buf, sem)

    def wait_gather(i, buf, sem):
        pltpu.make_async_copy(g_src(i), buf, sem).wait()

    def a_dst(i):
        return acc_s.at[ridx_buf.at[i]]

    def issue_scatter(i, buf, sem):
        pltpu.async_copy(buf, a_dst(i), sem, add=True)

    def wait_scatter(i, buf, sem):
        pltpu.make_async_copy(buf, a_dst(i), sem).wait()

    issue_gather(0, rows_a, sem_ga)
    issue_gather(1, rows_b, sem_gb)

    def body(j, _):
        i0 = 2 * j
        i1 = i0 + 1
        wait_gather(i0, rows_a, sem_ga)
        issue_scatter(i0, rows_a, sem_sa)

        @pl.when(i1 < NCH)
        def _():
            wait_gather(i1, rows_b, sem_gb)
            issue_scatter(i1, rows_b, sem_sb)

        @pl.when(i0 + 2 < NCH)
        def _():
            wait_scatter(i0, rows_a, sem_sa)
            issue_gather(i0 + 2, rows_a, sem_ga)

        @pl.when(i1 + 2 < NCH)
        def _():
            wait_scatter(i1, rows_b, sem_sb)
            issue_gather(i1 + 2, rows_b, sem_gb)

        return 0

    lax.fori_loop(0, (NCH + 1) // 2, body, 0)
    wait_scatter(NCH - 1, rows_a, sem_sa)
    wait_scatter(NCH - 2, rows_b, sem_sb)
    plsc.subcore_barrier()

    pltpu.sync_copy(acc_s.at[pl.ds(s * APT, APT)],
                    out_hbm.at[c, pl.ds(s * APT, APT)])


_agg_call = pl.kernel(
    _agg_body,
    out_type=jax.ShapeDtypeStruct((NC, NPAD, D), jnp.float32),
    mesh=_mesh,
    scratch_types=[
        pltpu.VMEM((EPW,), jnp.int32),
        pltpu.VMEM((NCH, K), jnp.int32),
        pltpu.VMEM((K, D), jnp.float32),
        pltpu.VMEM((K, D), jnp.float32),
        pltpu.VMEM_SHARED((NPAD, D), jnp.float32),
        pltpu.SemaphoreType.DMA,
        pltpu.SemaphoreType.DMA,
        pltpu.SemaphoreType.DMA,
        pltpu.SemaphoreType.DMA,
    ],
)


def _aggregate(g, row, col):
    row3 = row.reshape(NW, NCH, K)
    zeros = jnp.zeros((APT, D), jnp.float32)
    return _agg_call(g, col, row3, zeros)


_RB = 2000


def _linear_body(x_ref, w_ref, degp_ref, g_ref, dis_ref):
    deg = degp_ref[0, :, 0:1] + degp_ref[1, :, 0:1] + 2.0
    dis = lax.rsqrt(deg)
    h = jnp.dot(x_ref[...], w_ref[...], preferred_element_type=jnp.float32)
    g_ref[...] = dis * h
    dis_ref[...] = dis


def _linear(x, weight, deg_part):
    return pl.pallas_call(
        _linear_body,
        grid=(N // _RB,),
        in_specs=[
            pl.BlockSpec((_RB, D), lambda i: (i, 0)),
            pl.BlockSpec((D, D), lambda i: (0, 0)),
            pl.BlockSpec((NC, _RB, 128), lambda i: (0, i, 0)),
        ],
        out_specs=[
            pl.BlockSpec((_RB, D), lambda i: (i, 0)),
            pl.BlockSpec((_RB, 1), lambda i: (i, 0)),
        ],
        out_shape=[
            jax.ShapeDtypeStruct((N, D), jnp.float32),
            jax.ShapeDtypeStruct((N, 1), jnp.float32),
        ],
    )(x, weight, deg_part)


def _finish_body(acc_ref, g_ref, dis_ref, o_ref):
    acc = acc_ref[0] + acc_ref[1]
    o_ref[...] = jnp.maximum(dis_ref[...] * (acc + 2.0 * g_ref[...]), 0.0)


def _finish(acc, g, dis):
    return pl.pallas_call(
        _finish_body,
        grid=(N // _RB,),
        in_specs=[
            pl.BlockSpec((NC, _RB, D), lambda i: (0, i, 0)),
            pl.BlockSpec((_RB, D), lambda i: (i, 0)),
            pl.BlockSpec((_RB, 1), lambda i: (i, 0)),
        ],
        out_specs=pl.BlockSpec((_RB, D), lambda i: (i, 0)),
        out_shape=jax.ShapeDtypeStruct((N, D), jnp.float32),
    )(acc, g, dis)


def kernel(x, edge_index, weight):
    row = edge_index[0]
    col = edge_index[1]
    deg_part = _hist(row)
    g, dis = _linear(x, weight, deg_part)
    acc = _aggregate(g, row, col)
    out = _finish(acc, g, dis)
    return out


# speedup vs baseline: 34.3328x; 1.4590x over previous
import jax
import jax.numpy as jnp
from jax import lax
from jax.experimental import pallas as pl
from jax.experimental.pallas import tpu as pltpu
from jax.experimental.pallas import tpu_sc as plsc

N = 10000
E = 320000
D = 128

NC, NS = 2, 16
NW = NC * NS
NPAD = 10240
EPW = E // NW
K = 80
NCH = EPW // K
ROWS_PT = NPAD // NS
APT = NPAD // NS

_mesh = plsc.VectorSubcoreMesh(
    core_axis_name="c", subcore_axis_name="s", num_cores=NC, num_subcores=NS)


def _hist_body(row3_hbm, ones_hbm, zeros_hbm, out_hbm, ridx_buf, ones_v,
               deg_s, sem_a, sem_b):
    c = lax.axis_index("c")
    s = lax.axis_index("s")

    pltpu.sync_copy(zeros_hbm, deg_s.at[pl.ds(s * (NPAD // NS), NPAD // NS)])
    pltpu.sync_copy(ones_hbm, ones_v)
    pltpu.sync_copy(row3_hbm.at[c * NS + s], ridx_buf)
    plsc.subcore_barrier()

    def d_dst(i):
        return deg_s.at[ridx_buf.at[i]]

    def issue(i, sem):
        pltpu.async_copy(ones_v, d_dst(i), sem, add=True)

    def wait(i, sem):
        pltpu.make_async_copy(ones_v, d_dst(i), sem).wait()

    issue(0, sem_a)
    issue(1, sem_b)

    def body(j, _):
        i0 = 2 * j

        @pl.when(i0 + 2 < NCH)
        def _():
            wait(i0, sem_a)
            issue(i0 + 2, sem_a)

        @pl.when(i0 + 3 < NCH)
        def _():
            wait(i0 + 1, sem_b)
            issue(i0 + 3, sem_b)

        return 0

    lax.fori_loop(0, (NCH + 1) // 2, body, 0)
    wait(NCH - 1, sem_a)
    wait(NCH - 2, sem_b)
    plsc.subcore_barrier()

    pltpu.sync_copy(deg_s.at[pl.ds(s * (NPAD // NS), NPAD // NS)],
                    out_hbm.at[c, pl.ds(s * (NPAD // NS), NPAD // NS)])


_hist_call = pl.kernel(
    _hist_body,
    out_type=jax.ShapeDtypeStruct((NC, NPAD), jnp.float32),
    mesh=_mesh,
    scratch_types=[
        pltpu.VMEM((NCH, K), jnp.int32),
        pltpu.VMEM((K,), jnp.float32),
        pltpu.VMEM_SHARED((NPAD,), jnp.float32),
        pltpu.SemaphoreType.DMA,
        pltpu.SemaphoreType.DMA,
    ],
)


def _hist(row):
    row3 = row.reshape(NW, NCH, K)
    ones = jnp.ones((K,), jnp.float32)
    zeros = jnp.zeros((NPAD // NS,), jnp.float32)
    return _hist_call(row3, ones, zeros)


def _agg_body(g_hbm, col_hbm, row3_hbm, zeros_hbm, out_hbm, cidx_buf, ridx_buf,
              rows_a, rows_b, acc_s, sem_ga, sem_gb, sem_sa, sem_sb):
    c = lax.axis_index("c")
    s = lax.axis_index("s")
    w = c * NS + s

    pltpu.sync_copy(zeros_hbm, acc_s.at[pl.ds(s * APT, APT)])
    pltpu.sync_copy(col_hbm.at[pl.ds(w * EPW, EPW)], cidx_buf)
    pltpu.sync_copy(row3_hbm.at[w], ridx_buf)
    plsc.subcore_barrier()

    def g_src(i):
        return g_hbm.at[cidx_buf.at[pl.ds(i * K, K)]]

    def issue_gather(i, buf, sem):
        pltpu.async_copy(g_src(i), buf, sem)

    def wait_gather(i, buf, sem):
        pltpu.make_async_copy(g_src(i), buf, sem).wait()

    def a_dst(i):
        return acc_s.at[ridx_buf.at[i]]

    def issue_scatter(i, buf, sem):
        pltpu.async_copy(buf, a_dst(i), sem, add=True)

    def wait_scatter(i, buf, sem):
        pltpu.make_async_copy(buf, a_dst(i), sem).wait()

    issue_gather(0, rows_a, sem_ga)
    issue_gather(1, rows_b, sem_gb)

    def body(j, _):
        i0 = 2 * j
        i1 = i0 + 1
        wait_gather(i0, rows_a, sem_ga)
        issue_scatter(i0, rows_a, sem_sa)

        @pl.when(i1 < NCH)
        def _():
            wait_gather(i1, rows_b, sem_gb)
            issue_scatter(i1, rows_b, sem_sb)

        @pl.when(i0 + 2 < NCH)
        def _():
            wait_scatter(i0, rows_a, sem_sa)
            issue_gather(i0 + 2, rows_a, sem_ga)

        @pl.when(i1 + 2 < NCH)
        def _():
            wait_scatter(i1, rows_b, sem_sb)
            issue_gather(i1 + 2, rows_b, sem_gb)

        return 0

    lax.fori_loop(0, (NCH + 1) // 2, body, 0)
    wait_scatter(NCH - 1, rows_a, sem_sa)
    wait_scatter(NCH - 2, rows_b, sem_sb)
    plsc.subcore_barrier()

    pltpu.sync_copy(acc_s.at[pl.ds(s * APT, APT)],
                    out_hbm.at[c, pl.ds(s * APT, APT)])


_agg_call = pl.kernel(
    _agg_body,
    out_type=jax.ShapeDtypeStruct((NC, NPAD, D), jnp.float32),
    mesh=_mesh,
    scratch_types=[
        pltpu.VMEM((EPW,), jnp.int32),
        pltpu.VMEM((NCH, K), jnp.int32),
        pltpu.VMEM((K, D), jnp.float32),
        pltpu.VMEM((K, D), jnp.float32),
        pltpu.VMEM_SHARED((NPAD, D), jnp.float32),
        pltpu.SemaphoreType.DMA,
        pltpu.SemaphoreType.DMA,
        pltpu.SemaphoreType.DMA,
        pltpu.SemaphoreType.DMA,
    ],
)


def _aggregate(g, row, col):
    row3 = row.reshape(NW, NCH, K)
    zeros = jnp.zeros((APT, D), jnp.float32)
    return _agg_call(g, col, row3, zeros)


_RB = 2000


def _linear_body(x_ref, w_ref, degp_ref, g_ref, dis_ref):
    deg = degp_ref[0] + degp_ref[1] + 2.0
    dis = lax.rsqrt(deg)
    h = jnp.dot(x_ref[...], w_ref[...], preferred_element_type=jnp.float32)
    g_ref[...] = dis * h
    dis_ref[...] = dis


def _linear(x, weight, deg_part):
    return pl.pallas_call(
        _linear_body,
        grid=(N // _RB,),
        in_specs=[
            pl.BlockSpec((_RB, D), lambda i: (i, 0)),
            pl.BlockSpec((D, D), lambda i: (0, 0)),
            pl.BlockSpec((NC, _RB, 1), lambda i: (0, i, 0)),
        ],
        out_specs=[
            pl.BlockSpec((_RB, D), lambda i: (i, 0)),
            pl.BlockSpec((_RB, 1), lambda i: (i, 0)),
        ],
        out_shape=[
            jax.ShapeDtypeStruct((N, D), jnp.float32),
            jax.ShapeDtypeStruct((N, 1), jnp.float32),
        ],
    )(x, weight, deg_part.reshape(NC, NPAD, 1))


def _finish_body(acc_ref, g_ref, dis_ref, o_ref):
    acc = acc_ref[0] + acc_ref[1]
    o_ref[...] = jnp.maximum(dis_ref[...] * (acc + 2.0 * g_ref[...]), 0.0)


def _finish(acc, g, dis):
    return pl.pallas_call(
        _finish_body,
        grid=(N // _RB,),
        in_specs=[
            pl.BlockSpec((NC, _RB, D), lambda i: (0, i, 0)),
            pl.BlockSpec((_RB, D), lambda i: (i, 0)),
            pl.BlockSpec((_RB, 1), lambda i: (i, 0)),
        ],
        out_specs=pl.BlockSpec((_RB, D), lambda i: (i, 0)),
        out_shape=jax.ShapeDtypeStruct((N, D), jnp.float32),
    )(acc, g, dis)


def kernel(x, edge_index, weight):
    row = edge_index[0]
    col = edge_index[1]
    deg_part = _hist(row)
    g, dis = _linear(x, weight, deg_part)
    acc = _aggregate(g, row, col)
    out = _finish(acc, g, dis)
    return out
